# Initial kernel scaffold; baseline (speedup 1.0000x reference)
import jax
import jax.numpy as jnp
from jax.experimental import pallas as pl


def kernel(fbas_indices, time_step, sign, hour, day, month, day_of_week, is_weekend, fbas_count, emb, W1, b1, W2, b2, W3, b3):
    raise NotImplementedError("write your pallas kernel here")


# speedup vs baseline: 9.8579x; 9.8579x over previous
import functools

import jax
import jax.numpy as jnp
from jax import lax
from jax.experimental import pallas as pl
from jax.experimental.pallas import tpu as pltpu
from jax.experimental.pallas import tpu_sc as plsc

B = 16384
L = 200
EMB = 32

NC = 2
NS = 16
NW = NC * NS

ROWS_PER_W = B // NW
NB = 4
CHUNKS = ROWS_PER_W // NB
IDX_W = 100
GATHERS = NB * L // IDX_W
IDX_ROWS_PER_CHUNK = GATHERS


def _sc_body(idx_hbm, table_hbm, out_hbm, idx_v, buf, stage, sem):
    c = lax.axis_index("c")
    s = lax.axis_index("s")
    w = c * NS + s
    idx_base = w * (ROWS_PER_W * L // IDX_W)

    def chunk_body(g, carry):
        pltpu.sync_copy(
            idx_hbm.at[pl.ds(idx_base + g * IDX_ROWS_PER_CHUNK, IDX_ROWS_PER_CHUNK)],
            idx_v,
        )
        copies = [
            pltpu.async_copy(
                table_hbm.at[idx_v.at[k]],
                buf.at[pl.ds(k * IDX_W, IDX_W)],
                sem,
            )
            for k in range(GATHERS)
        ]
        for cp in copies:
            cp.wait()
        for r in range(NB):
            def acc_body(j, acc):
                a0, a1 = acc
                row = r * L + j
                return (a0 + buf[row, pl.ds(0, 16)], a1 + buf[row, pl.ds(16, 16)])
            a0, a1 = lax.fori_loop(
                0, L, acc_body,
                (jnp.zeros((16,), jnp.float32), jnp.zeros((16,), jnp.float32)),
            )
            out_row = g * NB + r
            stage[out_row, pl.ds(0, 16)] = a0
            stage[out_row, pl.ds(16, 16)] = a1
        return carry

    lax.fori_loop(0, CHUNKS, chunk_body, 0)
    pltpu.sync_copy(stage, out_hbm.at[pl.ds(w * ROWS_PER_W, ROWS_PER_W)])


def _sc_gather_sum(idx2d, emb):
    f = pl.kernel(
        _sc_body,
        out_type=jax.ShapeDtypeStruct((B, EMB), jnp.float32),
        mesh=plsc.VectorSubcoreMesh(core_axis_name="c", subcore_axis_name="s"),
        scratch_types=[
            pltpu.VMEM((IDX_ROWS_PER_CHUNK, IDX_W), jnp.int32),
            pltpu.VMEM((NB * L, EMB), jnp.float32),
            pltpu.VMEM((ROWS_PER_W, EMB), jnp.float32),
            pltpu.SemaphoreType.DMA,
        ],
        compiler_params=pltpu.CompilerParams(use_tc_tiling_on_sc=False),
    )
    return f(idx2d, emb)


BLK = 1024


def _mlp_body(scal_ref, pooled_ref, w1_ref, b1_ref, w2_ref, b2_ref, w3_ref, b3_ref, out_ref):
    xs = scal_ref[...]
    xe = pooled_ref[...] * (1.0 / L)
    h = (
        jnp.dot(xs, w1_ref[0:8, :], preferred_element_type=jnp.float32)
        + jnp.dot(xe, w1_ref[8:8 + EMB, :], preferred_element_type=jnp.float32)
        + b1_ref[...]
    )
    h = jnp.where(h > 0, h, 0.01 * h)
    h2 = jnp.dot(h, w2_ref[...], preferred_element_type=jnp.float32) + b2_ref[...]
    o = jnp.dot(h2, w3_ref[...], preferred_element_type=jnp.float32)[:, 0] + b3_ref[...]
    out_ref[0, :] = o


def _mlp(scal, pooled, W1, b1, W2, b2, W3, b3):
    grid = B // BLK
    full = lambda shape: pl.BlockSpec(shape, lambda i: tuple(0 for _ in shape))
    out = pl.pallas_call(
        _mlp_body,
        grid=(grid,),
        in_specs=[
            pl.BlockSpec((BLK, 8), lambda i: (i, 0)),
            pl.BlockSpec((BLK, EMB), lambda i: (i, 0)),
            full(W1.shape),
            full(b1.shape),
            full(W2.shape),
            full(b2.shape),
            full(W3.shape),
            full(b3.shape),
        ],
        out_specs=pl.BlockSpec((1, BLK), lambda i: (0, i)),
        out_shape=jax.ShapeDtypeStruct((1, B), jnp.float32),
    )(scal, pooled, W1, b1, W2, b2, W3, b3)
    return out.reshape(B)


def kernel(fbas_indices, time_step, sign, hour, day, month, day_of_week,
           is_weekend, fbas_count, emb, W1, b1, W2, b2, W3, b3):
    idx2d = fbas_indices.astype(jnp.int32).reshape(B * L // IDX_W, IDX_W)
    pooled = _sc_gather_sum(idx2d, emb)
    scal = jnp.stack(
        [time_step, sign, hour, day, month, day_of_week, is_weekend, fbas_count],
        axis=1,
    )
    return _mlp(scal, pooled, W1, b1, W2, b2, W3, b3)


# speedup vs baseline: 15.0538x; 1.5271x over previous
import functools

import jax
import jax.numpy as jnp
from jax import lax
from jax.experimental import pallas as pl
from jax.experimental.pallas import tpu as pltpu
from jax.experimental.pallas import tpu_sc as plsc

B = 16384
L = 200
EMB = 32

NC = 2
NS = 16
NW = NC * NS

ROWS_PER_W = B // NW
NB = 4
CHUNKS = ROWS_PER_W // NB
IDX_W = 100
GATHERS = NB * L // IDX_W
IDX_ROWS_PER_CHUNK = GATHERS


ACC_UNROLL = 8


def _sc_body(idx_hbm, table_hbm, out_hbm, idx_a, idx_b, buf_a, buf_b, stage,
             sem_g, sem_i):
    c = lax.axis_index("c")
    s = lax.axis_index("s")
    w = c * NS + s
    idx_base = w * (ROWS_PER_W * L // IDX_W)

    def idx_copy(g, idx_ref):
        return pltpu.make_async_copy(
            idx_hbm.at[pl.ds(idx_base + g * IDX_ROWS_PER_CHUNK, IDX_ROWS_PER_CHUNK)],
            idx_ref,
            sem_i,
        )

    def gather_copies(idx_ref, buf_ref):
        return [
            pltpu.make_async_copy(
                table_hbm.at[idx_ref.at[k]],
                buf_ref.at[pl.ds(k * IDX_W, IDX_W)],
                sem_g,
            )
            for k in range(GATHERS)
        ]

    pltpu.sync_copy(
        idx_hbm.at[pl.ds(idx_base, IDX_ROWS_PER_CHUNK)], idx_a)
    for cp in gather_copies(idx_a, buf_a):
        cp.start()
    idx_copy(1, idx_b).start()

    def half_body(g, idx_cur, buf_cur, idx_nxt, buf_nxt):
        for cp in gather_copies(idx_cur, buf_cur):
            cp.wait()
        @pl.when(g + 1 < CHUNKS)
        def _():
            idx_copy(g + 1, idx_nxt).wait()
            for cp in gather_copies(idx_nxt, buf_nxt):
                cp.start()
        @pl.when(g + 2 < CHUNKS)
        def _():
            idx_copy(g + 2, idx_cur).start()
        for r in range(NB):
            def acc_body(jo, acc):
                a0, a1 = acc
                for u in range(ACC_UNROLL):
                    row = r * L + jo * ACC_UNROLL + u
                    a0 = a0 + buf_cur[row, pl.ds(0, 16)]
                    a1 = a1 + buf_cur[row, pl.ds(16, 16)]
                return (a0, a1)
            a0, a1 = lax.fori_loop(
                0, L // ACC_UNROLL, acc_body,
                (jnp.zeros((16,), jnp.float32), jnp.zeros((16,), jnp.float32)),
            )
            out_row = g * NB + r
            stage[out_row, pl.ds(0, 16)] = a0
            stage[out_row, pl.ds(16, 16)] = a1

    def pair_body(go, carry):
        half_body(2 * go, idx_a, buf_a, idx_b, buf_b)
        half_body(2 * go + 1, idx_b, buf_b, idx_a, buf_a)
        return carry

    lax.fori_loop(0, CHUNKS // 2, pair_body, 0)
    pltpu.sync_copy(stage, out_hbm.at[pl.ds(w * ROWS_PER_W, ROWS_PER_W)])


def _sc_gather_sum(idx2d, emb):
    f = pl.kernel(
        _sc_body,
        out_type=jax.ShapeDtypeStruct((B, EMB), jnp.float32),
        mesh=plsc.VectorSubcoreMesh(core_axis_name="c", subcore_axis_name="s"),
        scratch_types=[
            pltpu.VMEM((IDX_ROWS_PER_CHUNK, IDX_W), jnp.int32),
            pltpu.VMEM((IDX_ROWS_PER_CHUNK, IDX_W), jnp.int32),
            pltpu.VMEM((NB * L, EMB), jnp.float32),
            pltpu.VMEM((NB * L, EMB), jnp.float32),
            pltpu.VMEM((ROWS_PER_W, EMB), jnp.float32),
            pltpu.SemaphoreType.DMA,
            pltpu.SemaphoreType.DMA,
        ],
        compiler_params=pltpu.CompilerParams(use_tc_tiling_on_sc=False),
    )
    return f(idx2d, emb)


BLK = 1024


def _mlp_body(scal_ref, pooled_ref, w1_ref, b1_ref, w2_ref, b2_ref, w3_ref, b3_ref, out_ref):
    xs = scal_ref[...]
    xe = pooled_ref[...] * (1.0 / L)
    h = (
        jnp.dot(xs, w1_ref[0:8, :], preferred_element_type=jnp.float32)
        + jnp.dot(xe, w1_ref[8:8 + EMB, :], preferred_element_type=jnp.float32)
        + b1_ref[...]
    )
    h = jnp.where(h > 0, h, 0.01 * h)
    h2 = jnp.dot(h, w2_ref[...], preferred_element_type=jnp.float32) + b2_ref[...]
    o = jnp.dot(h2, w3_ref[...], preferred_element_type=jnp.float32)[:, 0] + b3_ref[...]
    out_ref[0, :] = o


def _mlp(scal, pooled, W1, b1, W2, b2, W3, b3):
    grid = B // BLK
    full = lambda shape: pl.BlockSpec(shape, lambda i: tuple(0 for _ in shape))
    out = pl.pallas_call(
        _mlp_body,
        grid=(grid,),
        in_specs=[
            pl.BlockSpec((BLK, 8), lambda i: (i, 0)),
            pl.BlockSpec((BLK, EMB), lambda i: (i, 0)),
            full(W1.shape),
            full(b1.shape),
            full(W2.shape),
            full(b2.shape),
            full(W3.shape),
            full(b3.shape),
        ],
        out_specs=pl.BlockSpec((1, BLK), lambda i: (0, i)),
        out_shape=jax.ShapeDtypeStruct((1, B), jnp.float32),
    )(scal, pooled, W1, b1, W2, b2, W3, b3)
    return out.reshape(B)


def kernel(fbas_indices, time_step, sign, hour, day, month, day_of_week,
           is_weekend, fbas_count, emb, W1, b1, W2, b2, W3, b3):
    idx2d = fbas_indices.astype(jnp.int32).reshape(B * L // IDX_W, IDX_W)
    pooled = _sc_gather_sum(idx2d, emb)
    scal = jnp.stack(
        [time_step, sign, hour, day, month, day_of_week, is_weekend, fbas_count],
        axis=1,
    )
    return _mlp(scal, pooled, W1, b1, W2, b2, W3, b3)


# speedup vs baseline: 30.8291x; 2.0479x over previous
import jax
import jax.numpy as jnp
from jax import lax
from jax.experimental import pallas as pl
from jax.experimental.pallas import tpu as pltpu
from jax.experimental.pallas import tpu_sc as plsc

B = 16384
L = 200
EMB = 32
V = 1000000

NC = 2
NS = 16
NW = NC * NS

TK = 4
FULL_TILES = V // 128
TAIL = V - FULL_TILES * 128
STAGES = FULL_TILES // TK


def _tr_unit(buf, obuf, obp, k, lane33):
    @plsc.parallel_loop(0, 8, unroll=4)
    def v_body(vb):
        base = lane33 + (vb * (16 * 33))
        for e in range(EMB):
            x = buf[k, e, pl.ds(vb * 16, 16)]
            plsc.store_scatter(obp, [base + e], x)

    @plsc.parallel_loop(0, 128, unroll=8)
    def c_body(v):
        src = v * 33
        dst = k * (128 * EMB) + v * EMB
        obuf[pl.ds(dst, 16)] = obp[pl.ds(src, 16)]
        obuf[pl.ds(dst + 16, 16)] = obp[pl.ds(src + 16, 16)]


def _detile_body(src, src_tail, out, buf_a, buf_b, ob_a, ob_b, obp, tail_v, sem_i, sem_o):
    c = lax.axis_index("c")
    s = lax.axis_index("s")
    w = c * NS + s
    n_stages = jnp.where(w < STAGES % NW, STAGES // NW + 1, STAGES // NW)
    lane33 = lax.iota(jnp.int32, 16) * 33

    def in_copies(q, buf):
        cps = []
        for k in range(TK):
            for a in range(4):
                cps.append(pltpu.make_async_copy(
                    src.at[pl.ds(8 * a, 8), pl.ds((q * TK + k) * 128, 128)],
                    buf.at[k, pl.ds(8 * a, 8), :],
                    sem_i,
                ))
        return cps

    def out_copy(q, obuf):
        return pltpu.make_async_copy(
            obuf, out.at[pl.ds(q * (TK * 128 * EMB), TK * 128 * EMB)], sem_o)

    def stage_q(i):
        return w + i * NW

    for cp in in_copies(stage_q(0), buf_a):
        cp.start()

    def half(i, buf_cur, ob_cur, buf_nxt):
        q = stage_q(i)
        for cp in in_copies(q, buf_cur):
            cp.wait()
        @pl.when(i + 1 < n_stages)
        def _():
            for cp in in_copies(stage_q(i + 1), buf_nxt):
                cp.start()
        @pl.when(i >= 2)
        def _():
            out_copy(stage_q(i - 2), ob_cur).wait()
        for k in range(TK):
            _tr_unit(buf_cur, ob_cur, obp, k, lane33)
        out_copy(q, ob_cur).start()

    def pair(io, carry):
        i = 2 * io
        @pl.when(i < n_stages)
        def _():
            half(i, buf_a, ob_a, buf_b)
        @pl.when(i + 1 < n_stages)
        def _():
            half(i + 1, buf_b, ob_b, buf_a)
        return carry

    lax.fori_loop(0, (STAGES // NW + 2) // 2, pair, 0)
    @pl.when(n_stages >= 2)
    def _():
        out_copy(stage_q(n_stages - 2), ob_a).wait()
    @pl.when(n_stages >= 1)
    def _():
        out_copy(stage_q(n_stages - 1), ob_b).wait()

    @pl.when(w == 0)
    def _():
        pltpu.sync_copy(src_tail, tail_v)
        pltpu.sync_copy(tail_v, out.at[pl.ds(FULL_TILES * 128 * EMB, TAIL * EMB)])


def _detile_table(emb_t, emb_tail):
    f = pl.kernel(
        _detile_body,
        out_type=jax.ShapeDtypeStruct((V * EMB,), jnp.float32),
        mesh=plsc.VectorSubcoreMesh(core_axis_name="c", subcore_axis_name="s"),
        scratch_types=[
            pltpu.VMEM((TK, EMB, 128), jnp.float32),
            pltpu.VMEM((TK, EMB, 128), jnp.float32),
            pltpu.VMEM((TK * 128 * EMB,), jnp.float32),
            pltpu.VMEM((TK * 128 * EMB,), jnp.float32),
            pltpu.VMEM((128 * 33,), jnp.float32),
            pltpu.VMEM((TAIL * EMB,), jnp.float32),
            pltpu.SemaphoreType.DMA,
            pltpu.SemaphoreType.DMA,
        ],
        compiler_params=pltpu.CompilerParams(use_tc_tiling_on_sc=True, needs_layout_passes=False),
    )
    return f(emb_t, emb_tail)


ROWS_PER_W = B // NW
NB = 8
CHUNKS = ROWS_PER_W // NB
IDX_SPLITS = ((0, 120), (120, 80))
ACC_UNROLL = 8


def _sc_body(idx_hbm, table_hbm, out_hbm, idx_a, idx_b, buf_a, buf_b, stage,
             sem_g, sem_i):
    c = lax.axis_index("c")
    s = lax.axis_index("s")
    w = c * NS + s
    row_base = w * ROWS_PER_W

    def idx_copy(g, idx_ref):
        return pltpu.make_async_copy(
            idx_hbm.at[pl.ds(row_base + g * NB, NB), :], idx_ref, sem_i)

    def gather_copies(idx_ref, buf_ref):
        cps = []
        for r in range(NB):
            for off, n in IDX_SPLITS:
                cps.append(pltpu.make_async_copy(
                    table_hbm.at[idx_ref.at[r, pl.ds(off, n)]],
                    buf_ref.at[pl.ds(r * L + off, n)],
                    sem_g,
                ))
        return cps

    pltpu.sync_copy(idx_hbm.at[pl.ds(row_base, NB), :], idx_a)
    for cp in gather_copies(idx_a, buf_a):
        cp.start()
    idx_copy(1, idx_b).start()

    def half_body(g, idx_cur, buf_cur, idx_nxt, buf_nxt):
        for cp in gather_copies(idx_cur, buf_cur):
            cp.wait()
        @pl.when(g + 1 < CHUNKS)
        def _():
            idx_copy(g + 1, idx_nxt).wait()
            for cp in gather_copies(idx_nxt, buf_nxt):
                cp.start()
        @pl.when(g + 2 < CHUNKS)
        def _():
            idx_copy(g + 2, idx_cur).start()
        for r in range(NB):
            def acc_body(jo, acc):
                a0, a1 = acc
                for u in range(ACC_UNROLL):
                    row = r * L + jo * ACC_UNROLL + u
                    a0 = a0 + buf_cur[row, pl.ds(0, 16)]
                    a1 = a1 + buf_cur[row, pl.ds(16, 16)]
                return (a0, a1)
            a0, a1 = lax.fori_loop(
                0, L // ACC_UNROLL, acc_body,
                (jnp.zeros((16,), jnp.float32), jnp.zeros((16,), jnp.float32)),
            )
            out_row = g * NB + r
            stage[out_row, pl.ds(0, 16)] = a0
            stage[out_row, pl.ds(16, 16)] = a1

    def pair_body(go, carry):
        half_body(2 * go, idx_a, buf_a, idx_b, buf_b)
        half_body(2 * go + 1, idx_b, buf_b, idx_a, buf_a)
        return carry

    lax.fori_loop(0, CHUNKS // 2, pair_body, 0)
    pltpu.sync_copy(stage, out_hbm.at[pl.ds(row_base, ROWS_PER_W)])


def _sc_gather_sum(idx, table):
    f = pl.kernel(
        _sc_body,
        out_type=jax.ShapeDtypeStruct((B, EMB), jnp.float32),
        mesh=plsc.VectorSubcoreMesh(core_axis_name="c", subcore_axis_name="s"),
        scratch_types=[
            pltpu.VMEM((NB, L), jnp.int32),
            pltpu.VMEM((NB, L), jnp.int32),
            pltpu.VMEM((NB * L, EMB), jnp.float32),
            pltpu.VMEM((NB * L, EMB), jnp.float32),
            pltpu.VMEM((ROWS_PER_W, EMB), jnp.float32),
            pltpu.SemaphoreType.DMA,
            pltpu.SemaphoreType.DMA,
        ],
        compiler_params=pltpu.CompilerParams(use_tc_tiling_on_sc=False),
    )
    return f(idx, table)


BLK = 1024


def _mlp_body(scal_ref, pooled_ref, w1_ref, b1_ref, w2_ref, b2_ref, w3_ref, b3_ref, out_ref):
    xs = scal_ref[...]
    xe = pooled_ref[...] * (1.0 / L)
    h = (
        jnp.dot(xs, w1_ref[0:8, :], preferred_element_type=jnp.float32)
        + jnp.dot(xe, w1_ref[8:8 + EMB, :], preferred_element_type=jnp.float32)
        + b1_ref[...]
    )
    h = jnp.where(h > 0, h, 0.01 * h)
    h2 = jnp.dot(h, w2_ref[...], preferred_element_type=jnp.float32) + b2_ref[...]
    o = jnp.dot(h2, w3_ref[...], preferred_element_type=jnp.float32)[:, 0] + b3_ref[...]
    out_ref[0, :] = o


def _mlp(scal, pooled, W1, b1, W2, b2, W3, b3):
    grid = B // BLK
    full = lambda shape: pl.BlockSpec(shape, lambda i: tuple(0 for _ in shape))
    out = pl.pallas_call(
        _mlp_body,
        grid=(grid,),
        in_specs=[
            pl.BlockSpec((BLK, 8), lambda i: (i, 0)),
            pl.BlockSpec((BLK, EMB), lambda i: (i, 0)),
            full(W1.shape),
            full(b1.shape),
            full(W2.shape),
            full(b2.shape),
            full(W3.shape),
            full(b3.shape),
        ],
        out_specs=pl.BlockSpec((1, BLK), lambda i: (0, i)),
        out_shape=jax.ShapeDtypeStruct((1, B), jnp.float32),
    )(scal, pooled, W1, b1, W2, b2, W3, b3)
    return out.reshape(B)


def kernel(fbas_indices, time_step, sign, hour, day, month, day_of_week,
           is_weekend, fbas_count, emb, W1, b1, W2, b2, W3, b3):
    emb_tail = emb[FULL_TILES * 128:, :].reshape(TAIL * EMB)
    table = _detile_table(emb.T, emb_tail).reshape(V, EMB)
    pooled = _sc_gather_sum(fbas_indices.astype(jnp.int32), table)
    scal = jnp.stack(
        [time_step, sign, hour, day, month, day_of_week, is_weekend, fbas_count],
        axis=1,
    )
    return _mlp(scal, pooled, W1, b1, W2, b2, W3, b3)
